# baseline (speedup 1.0000x reference)
import functools

import jax
import jax.numpy as jnp
from jax import lax
from jax.experimental import pallas as pl
from jax.experimental.pallas import tpu as pltpu
from jax.experimental.pallas import tpu_sc as plsc

NUM_CORES = 2
NUM_SUBCORES = 16
NW = NUM_CORES * NUM_SUBCORES


@functools.partial(jax.jit, static_argnames=("n", "d", "c"))
def _gather_sc(idx, encoding, n, d, c):
    b_per_w = n // NW
    n_chunks = b_per_w // c
    mesh = plsc.VectorSubcoreMesh(core_axis_name="c", subcore_axis_name="s")

    @functools.partial(
        pl.kernel,
        mesh=mesh,
        out_type=jax.ShapeDtypeStruct((n, d), jnp.float32),
        scratch_types=[
            pltpu.VMEM((n_chunks, c), jnp.int32),
            pltpu.VMEM((c, d), jnp.float32),
            pltpu.VMEM((c, d), jnp.float32),
            pltpu.VMEM((c, d), jnp.float32),
            pltpu.VMEM((c, d), jnp.float32),
            pltpu.SemaphoreType.DMA,
            pltpu.SemaphoreType.DMA,
            pltpu.SemaphoreType.DMA,
            pltpu.SemaphoreType.DMA,
            pltpu.SemaphoreType.DMA,
            pltpu.SemaphoreType.DMA,
            pltpu.SemaphoreType.DMA,
            pltpu.SemaphoreType.DMA,
        ],
    )
    def k(enc_hbm, idx_hbm, out_hbm, idx_v, b0, b1, b2, b3,
          g0, g1, g2, g3, w0, w1, w2, w3):
        wid = lax.axis_index("s") * NUM_CORES + lax.axis_index("c")
        base = wid * b_per_w

        pltpu.sync_copy(idx_hbm.at[wid], idx_v)

        bufs = (b0, b1, b2, b3)
        gsems = (g0, g1, g2, g3)
        wsems = (w0, w1, w2, w3)

        def gather_start(j, slot):
            pltpu.async_copy(enc_hbm.at[idx_v.at[j]], bufs[slot], gsems[slot])

        def gather_wait(j, slot):
            pltpu.make_async_copy(
                enc_hbm.at[idx_v.at[j]], bufs[slot], gsems[slot]
            ).wait()

        def write_start(j, slot):
            pltpu.async_copy(
                bufs[slot], out_hbm.at[pl.ds(base + j * c, c)], wsems[slot]
            )

        def write_wait(j, slot):
            pltpu.make_async_copy(
                bufs[slot], out_hbm.at[pl.ds(base + j * c, c)], wsems[slot]
            ).wait()

        gather_start(0, 0)
        gather_start(1, 1)
        gather_wait(0, 0)
        write_start(0, 0)
        gather_start(2, 2)
        gather_wait(1, 1)
        write_start(1, 1)
        gather_start(3, 3)

        def body(g, carry):
            j0 = 2 + 4 * g
            for i in range(4):
                j = j0 + i
                slot = (2 + i) % 4
                gather_wait(j, slot)
                write_start(j, slot)
                write_wait(j - 2, (slot + 2) % 4)
                gather_start(j + 2, (slot + 2) % 4)
            return carry

        lax.fori_loop(0, (n_chunks - 4) // 4, body, 0)

        j = n_chunks - 2
        gather_wait(j, j % 4)
        write_start(j, j % 4)
        write_wait(j - 2, (j + 2) % 4)
        gather_wait(j + 1, (j + 1) % 4)
        write_start(j + 1, (j + 1) % 4)
        write_wait(j - 1, (j + 3) % 4)
        write_wait(j, j % 4)
        write_wait(j + 1, (j + 1) % 4)

    return k(encoding, idx)


def kernel(x, encoding):
    b, s = x.shape
    v, d = encoding.shape
    n = b * s
    c = 16
    n_sc = (n * 3 // 4 // (NW * c)) * (NW * c)
    xf = x.reshape(n).astype(jnp.int32)
    idx_sc = xf[:n_sc].reshape(NW, (n_sc // NW) // c, c)
    out_sc = _gather_sc(idx_sc, encoding, n_sc, d, c)
    out_tc = jnp.take(encoding, xf[n_sc:], axis=0)
    return jnp.concatenate([out_sc, out_tc], axis=0).reshape(b, s, d)


# speedup vs baseline: 1.9560x; 1.9560x over previous
import functools

import jax
import jax.numpy as jnp
from jax import lax
from jax.experimental import pallas as pl
from jax.experimental.pallas import tpu as pltpu
from jax.experimental.pallas import tpu_sc as plsc

NUM_CORES = 2
NUM_SUBCORES = 16
NW = NUM_CORES * NUM_SUBCORES

C = 32
S = 3
G = 2


@functools.partial(jax.jit, static_argnames=("n", "d"))
def _gather_sc(idx, encoding, n, d):
    b_per_w = n // NW
    n_chunks = b_per_w // C
    mesh = plsc.VectorSubcoreMesh(core_axis_name="c", subcore_axis_name="s")

    @functools.partial(
        pl.kernel,
        mesh=mesh,
        out_type=jax.ShapeDtypeStruct((n, d), jnp.float32),
        scratch_types=(
            [pltpu.VMEM((n_chunks, C), jnp.int32)]
            + [pltpu.VMEM((C, d), jnp.float32) for _ in range(S)]
            + [pltpu.SemaphoreType.DMA for _ in range(2 * S)]
        ),
    )
    def k(enc_hbm, idx_hbm, out_hbm, idx_v, *bufsems):
        bufs = bufsems[:S]
        gsems = bufsems[S:2 * S]
        wsems = bufsems[2 * S:]
        wid = lax.axis_index("s") * NUM_CORES + lax.axis_index("c")
        base = wid * b_per_w

        pltpu.sync_copy(idx_hbm.at[wid], idx_v)

        def gather_start(j, slot):
            pltpu.async_copy(enc_hbm.at[idx_v.at[j]], bufs[slot], gsems[slot])

        def gather_wait(j, slot):
            pltpu.make_async_copy(
                enc_hbm.at[idx_v.at[j]], bufs[slot], gsems[slot]
            ).wait()

        def write_start(j, slot):
            pltpu.async_copy(
                bufs[slot], out_hbm.at[pl.ds(base + j * C, C)], wsems[slot]
            )

        def write_wait(j, slot):
            pltpu.make_async_copy(
                bufs[slot], out_hbm.at[pl.ds(base + j * C, C)], wsems[slot]
            ).wait()

        def step(j, slot, with_wait_w, with_start_g):
            gather_wait(j, slot)
            write_start(j, slot)
            if with_wait_w:
                write_wait(j - (S - G), (slot + G) % S)
            if with_start_g:
                gather_start(j + G, (slot + G) % S)

        for g in range(G):
            gather_start(g, g % S)

        lo = S - G
        hi = n_chunks - G
        for j in range(lo):
            step(j, j % S, False, True)

        n_main = hi - lo
        n_groups = n_main // S

        def body(gi, carry):
            j0 = lo + S * gi
            for i in range(S):
                step(j0 + i, (lo + i) % S, True, True)
            return carry

        lax.fori_loop(0, n_groups, body, 0)
        for j in range(lo + n_groups * S, hi):
            step(j, j % S, True, True)

        for j in range(hi, n_chunks):
            step(j, j % S, True, False)
        for j in range(n_chunks - (S - G), n_chunks):
            write_wait(j, j % S)

    return k(encoding, idx)


def kernel(x, encoding):
    b, s = x.shape
    v, d = encoding.shape
    n = b * s
    idx = x.reshape(NW, (n // NW) // C, C).astype(jnp.int32)
    out = _gather_sc(idx, encoding, n, d)
    return out.reshape(b, s, d)


# speedup vs baseline: 1.9735x; 1.0090x over previous
import functools

import jax
import jax.numpy as jnp
from jax import lax
from jax.experimental import pallas as pl
from jax.experimental.pallas import tpu as pltpu
from jax.experimental.pallas import tpu_sc as plsc

NUM_CORES = 2
NUM_SUBCORES = 16
NW = NUM_CORES * NUM_SUBCORES

C = 16
S = 6
G = 4


@functools.partial(jax.jit, static_argnames=("n", "d"))
def _gather_sc(idx, encoding, n, d):
    b_per_w = n // NW
    n_chunks = b_per_w // C
    mesh = plsc.VectorSubcoreMesh(core_axis_name="c", subcore_axis_name="s")

    @functools.partial(
        pl.kernel,
        mesh=mesh,
        out_type=jax.ShapeDtypeStruct((n, d), jnp.float32),
        scratch_types=(
            [pltpu.VMEM((n_chunks, C), jnp.int32)]
            + [pltpu.VMEM((C, d), jnp.float32) for _ in range(S)]
            + [pltpu.SemaphoreType.DMA for _ in range(2 * S)]
        ),
    )
    def k(enc_hbm, idx_hbm, out_hbm, idx_v, *bufsems):
        bufs = bufsems[:S]
        gsems = bufsems[S:2 * S]
        wsems = bufsems[2 * S:]
        wid = lax.axis_index("s") * NUM_CORES + lax.axis_index("c")
        base = wid * b_per_w

        pltpu.sync_copy(idx_hbm.at[wid], idx_v)

        def gather_start(j, slot):
            pltpu.async_copy(enc_hbm.at[idx_v.at[j]], bufs[slot], gsems[slot])

        def gather_wait(j, slot):
            pltpu.make_async_copy(
                enc_hbm.at[idx_v.at[j]], bufs[slot], gsems[slot]
            ).wait()

        def write_start(j, slot):
            pltpu.async_copy(
                bufs[slot], out_hbm.at[pl.ds(base + j * C, C)], wsems[slot]
            )

        def write_wait(j, slot):
            pltpu.make_async_copy(
                bufs[slot], out_hbm.at[pl.ds(base + j * C, C)], wsems[slot]
            ).wait()

        def step(j, slot, with_wait_w, with_start_g):
            gather_wait(j, slot)
            write_start(j, slot)
            if with_wait_w:
                write_wait(j - (S - G), (slot + G) % S)
            if with_start_g:
                gather_start(j + G, (slot + G) % S)

        for g in range(G):
            gather_start(g, g % S)

        lo = S - G
        hi = n_chunks - G
        for j in range(lo):
            step(j, j % S, False, True)

        n_main = hi - lo
        n_groups = n_main // S

        def body(gi, carry):
            j0 = lo + S * gi
            for i in range(S):
                step(j0 + i, (lo + i) % S, True, True)
            return carry

        lax.fori_loop(0, n_groups, body, 0)
        for j in range(lo + n_groups * S, hi):
            step(j, j % S, True, True)

        for j in range(hi, n_chunks):
            step(j, j % S, True, False)
        for j in range(n_chunks - (S - G), n_chunks):
            write_wait(j, j % S)

    return k(encoding, idx)


def kernel(x, encoding):
    b, s = x.shape
    v, d = encoding.shape
    n = b * s
    idx = x.reshape(NW, (n // NW) // C, C).astype(jnp.int32)
    out = _gather_sc(idx, encoding, n, d)
    return out.reshape(b, s, d)
